# baseline (speedup 1.0000x reference)
import functools

import jax
import jax.numpy as jnp
from jax import lax
from jax.experimental import pallas as pl
from jax.experimental.pallas import tpu as pltpu
from jax.experimental.pallas import tpu_sc as plsc

BB = 256
L = 16


@functools.lru_cache(maxsize=None)
def _build_gather(BATCH, H, V, D, dtype_name):
    dtype = jnp.dtype(dtype_name)
    info = plsc.get_sparse_core_info()
    NC, NS = info.num_cores, info.num_subcores
    NW = NC * NS
    NF = D // 8
    NBQ = BB // 128
    n_units = H * (BATCH // BB)
    assert n_units % (2 * NW) == 0
    u_per_w = n_units // NW
    nbb = BATCH // BB
    mesh = plsc.VectorSubcoreMesh(core_axis_name="c", subcore_axis_name="s")

    @functools.partial(
        pl.kernel,
        mesh=mesh,
        out_type=jax.ShapeDtypeStruct((H, NF, BATCH // 128, 8, 128), dtype),
        compiler_params=pltpu.CompilerParams(
            use_tc_tiling_on_sc=False, needs_layout_passes=False),
        scratch_types=[
            pltpu.VMEM((BB,), jnp.int32),
            pltpu.VMEM((BB,), jnp.int32),
            pltpu.VMEM((BB, 2 * D), dtype),
            pltpu.VMEM((BB, 2 * D), dtype),
            pltpu.VMEM((NF, NBQ, 8, 128), dtype),
            pltpu.VMEM((NF, NBQ, 8, 128), dtype),
            pltpu.SemaphoreType.DMA,
            pltpu.SemaphoreType.DMA,
            pltpu.SemaphoreType.DMA,
            pltpu.SemaphoreType.DMA,
        ],
    )
    def k(idx_hbm, table_hbm, out_hbm, idxu_a, idxu_b, rows_a, rows_b,
          sel_a, sel_b, gsem_a, gsem_b, osem_a, osem_b):
        wid = lax.axis_index("s") * NC + lax.axis_index("c")
        u0 = wid * u_per_w

        def unit_hb(u):
            return u // nbb, lax.rem(u, nbb)

        def stage(u, idxu):
            h, bb = unit_hb(u)
            pltpu.sync_copy(idx_hbm.at[h, pl.ds(bb * BB, BB)], idxu)

        def gather(idxu, rows, sem):
            pltpu.async_copy(table_hbm.at[idxu], rows, sem)

        def wait_gather(idxu, rows, sem):
            pltpu.make_async_copy(table_hbm.at[idxu], rows, sem).wait()

        def pack(rows, sel):
            iota = lax.iota(jnp.int32, L)

            def body(t, _):
                bq = t // 8
                g16 = lax.rem(t, 8) * L
                b16 = iota + t * L
                for F in range(NF):
                    for f in range(8):
                        col = iota * 0 + (F * 8 + f)
                        x = plsc.load_gather(rows, [b16, col])
                        sel[F, bq, f, pl.ds(g16, L)] = x
                return 0

            lax.fori_loop(0, NBQ * 8, body, 0)

        def put(u, sel, sem):
            h, bb = unit_hb(u)
            for F in range(NF):
                pltpu.async_copy(sel.at[F],
                                 out_hbm.at[h, F, pl.ds(bb * NBQ, NBQ)], sem)

        def wait_put(u, sel, sem):
            h, bb = unit_hb(u)
            for F in range(NF):
                pltpu.make_async_copy(sel.at[F],
                                      out_hbm.at[h, F,
                                                 pl.ds(bb * NBQ, NBQ)],
                                      sem).wait()

        stage(u0, idxu_a)
        gather(idxu_a, rows_a, gsem_a)
        stage(u0 + 1, idxu_b)
        gather(idxu_b, rows_b, gsem_b)

        def body(p, _):
            ua = u0 + 2 * p
            ub = ua + 1
            wait_gather(idxu_a, rows_a, gsem_a)
            pack(rows_a, sel_a)
            put(ua, sel_a, osem_a)
            wait_put(ua, sel_a, osem_a)

            @pl.when(2 * p + 2 < u_per_w)
            def _():
                stage(ua + 2, idxu_a)
                gather(idxu_a, rows_a, gsem_a)

            wait_gather(idxu_b, rows_b, gsem_b)
            pack(rows_b, sel_b)
            put(ub, sel_b, osem_b)
            wait_put(ub, sel_b, osem_b)

            @pl.when(2 * p + 3 < u_per_w)
            def _():
                stage(ub + 2, idxu_b)
                gather(idxu_b, rows_b, gsem_b)

            return 0

        lax.fori_loop(0, u_per_w // 2, body, 0)

    return k


@jax.jit
def kernel(input_sequences, weight):
    batch, hist = input_sequences.shape
    vocab, dim = weight.shape
    idx_t = input_sequences.T.astype(jnp.int32)
    table_p = jnp.pad(weight, ((0, 0), (0, dim)))
    fn = _build_gather(batch, hist, vocab, dim, weight.dtype.name)
    out5 = fn(idx_t, table_p)
    return jnp.transpose(out5, (2, 4, 0, 1, 3)).reshape(batch, hist, dim)


# speedup vs baseline: 1.5307x; 1.5307x over previous
import functools

import jax
import jax.numpy as jnp
from jax import lax
from jax.experimental import pallas as pl
from jax.experimental.pallas import tpu as pltpu
from jax.experimental.pallas import tpu_sc as plsc

S = 512


@functools.lru_cache(maxsize=None)
def _build_gather(B, V, D, dtype_name):
    dtype = jnp.dtype(dtype_name)
    info = plsc.get_sparse_core_info()
    NC, NS = info.num_cores, info.num_subcores
    NW = NC * NS
    assert B % (NW * S) == 0
    b_per_w = B // NW
    n_sc = b_per_w // S
    assert n_sc % 2 == 0
    mesh = plsc.VectorSubcoreMesh(core_axis_name="c", subcore_axis_name="s")

    @functools.partial(
        pl.kernel,
        mesh=mesh,
        out_type=jax.ShapeDtypeStruct((B, D), dtype),
        compiler_params=pltpu.CompilerParams(use_tc_tiling_on_sc=False),
        scratch_types=[
            pltpu.VMEM((b_per_w,), jnp.int32),
            pltpu.VMEM((S, D), dtype),
            pltpu.VMEM((S, D), dtype),
            pltpu.SemaphoreType.DMA,
            pltpu.SemaphoreType.DMA,
            pltpu.SemaphoreType.DMA,
            pltpu.SemaphoreType.DMA,
        ],
    )
    def k(idx_hbm, table_hbm, out_hbm, idx_v, rows_a, rows_b,
          gsem_a, gsem_b, osem_a, osem_b):
        wid = lax.axis_index("s") * NC + lax.axis_index("c")
        r0 = wid * b_per_w
        pltpu.sync_copy(idx_hbm.at[pl.ds(r0, b_per_w)], idx_v)

        def gather(sc, rows, sem):
            pltpu.async_copy(table_hbm.at[idx_v.at[pl.ds(sc * S, S)]],
                             rows, sem)

        def wait_gather(rows, sem):
            pltpu.make_async_copy(table_hbm.at[idx_v.at[pl.ds(0, S)]],
                                  rows, sem).wait()

        def put(sc, rows, sem):
            pltpu.async_copy(rows, out_hbm.at[pl.ds(r0 + sc * S, S)], sem)

        def wait_put(sc, rows, sem):
            pltpu.make_async_copy(rows, out_hbm.at[pl.ds(r0 + sc * S, S)],
                                  sem).wait()

        gather(0, rows_a, gsem_a)
        gather(1, rows_b, gsem_b)

        def body(p, _):
            sc_a = 2 * p
            sc_b = 2 * p + 1
            wait_gather(rows_a, gsem_a)
            put(sc_a, rows_a, osem_a)
            wait_put(sc_a, rows_a, osem_a)

            @pl.when(sc_a + 2 < n_sc)
            def _():
                gather(sc_a + 2, rows_a, gsem_a)

            wait_gather(rows_b, gsem_b)
            put(sc_b, rows_b, osem_b)
            wait_put(sc_b, rows_b, osem_b)

            @pl.when(sc_b + 2 < n_sc)
            def _():
                gather(sc_b + 2, rows_b, gsem_b)

            return 0

        lax.fori_loop(0, n_sc // 2, body, 0)

    return k


@jax.jit
def kernel(input_sequences, weight):
    batch, hist = input_sequences.shape
    vocab, dim = weight.shape
    idx = input_sequences.reshape(-1).astype(jnp.int32)
    fn = _build_gather(batch * hist, vocab, dim, weight.dtype.name)
    out = fn(idx, weight)
    return out.reshape(batch, hist, dim)


# speedup vs baseline: 2.4200x; 1.5810x over previous
import functools

import jax
import jax.numpy as jnp
from jax import lax
from jax.experimental import pallas as pl
from jax.experimental.pallas import tpu as pltpu
from jax.experimental.pallas import tpu_sc as plsc

BB = 256
L = 16


@functools.lru_cache(maxsize=None)
def _build_gather(BATCH, H, V, D, dtype_name):
    dtype = jnp.dtype(dtype_name)
    info = plsc.get_sparse_core_info()
    NC, NS = info.num_cores, info.num_subcores
    NW = NC * NS
    NF = D // 8
    NBQ = BB // 128
    n_units = H * (BATCH // BB)
    assert n_units % (2 * NW) == 0
    u_per_w = n_units // NW
    nbb = BATCH // BB
    mesh = plsc.VectorSubcoreMesh(core_axis_name="c", subcore_axis_name="s")

    @functools.partial(
        pl.kernel,
        mesh=mesh,
        out_type=jax.ShapeDtypeStruct((H, NF, BATCH // 128, 8, 128), dtype),
        compiler_params=pltpu.CompilerParams(
            use_tc_tiling_on_sc=False, needs_layout_passes=False),
        scratch_types=[
            pltpu.VMEM((BB,), jnp.int32),
            pltpu.VMEM((BB,), jnp.int32),
            pltpu.VMEM((BB, 2 * D), dtype),
            pltpu.VMEM((BB, 2 * D), dtype),
            pltpu.VMEM((NF, NBQ, 8, 128), dtype),
            pltpu.VMEM((NF, NBQ, 8, 128), dtype),
            pltpu.SemaphoreType.DMA,
            pltpu.SemaphoreType.DMA,
            pltpu.SemaphoreType.DMA,
            pltpu.SemaphoreType.DMA,
        ],
    )
    def k(idx_hbm, table_hbm, out_hbm, idxu_a, idxu_b, rows_a, rows_b,
          sel_a, sel_b, gsem_a, gsem_b, osem_a, osem_b):
        wid = lax.axis_index("s") * NC + lax.axis_index("c")
        u0 = wid * u_per_w

        def unit_hb(u):
            return u // nbb, lax.rem(u, nbb)

        def stage(u, idxu):
            h, bb = unit_hb(u)
            pltpu.sync_copy(idx_hbm.at[h, pl.ds(bb * BB, BB)], idxu)

        def gather(idxu, rows, sem):
            pltpu.async_copy(table_hbm.at[idxu], rows, sem)

        def wait_gather(idxu, rows, sem):
            pltpu.make_async_copy(table_hbm.at[idxu], rows, sem).wait()

        def pack(rows, sel):
            iota = lax.iota(jnp.int32, L)
            for bq in range(NBQ):
                @functools.partial(plsc.parallel_loop, 0, 8, unroll=2)
                def body(g, _bq=bq):
                    g16 = g * L
                    b16 = iota + (_bq * 128 + g16)
                    for F in range(NF):
                        for f in range(8):
                            col = iota * 0 + (F * 8 + f)
                            x = plsc.load_gather(rows, [b16, col])
                            sel[F, _bq, f, pl.ds(g16, L)] = x

        def put(u, sel, sem):
            h, bb = unit_hb(u)
            for F in range(NF):
                pltpu.async_copy(sel.at[F],
                                 out_hbm.at[h, F, pl.ds(bb * NBQ, NBQ)], sem)

        def wait_put(u, sel, sem):
            h, bb = unit_hb(u)
            for F in range(NF):
                pltpu.make_async_copy(sel.at[F],
                                      out_hbm.at[h, F,
                                                 pl.ds(bb * NBQ, NBQ)],
                                      sem).wait()

        stage(u0, idxu_a)
        gather(idxu_a, rows_a, gsem_a)
        stage(u0 + 1, idxu_b)
        gather(idxu_b, rows_b, gsem_b)

        def body(p, _):
            ua = u0 + 2 * p
            ub = ua + 1
            wait_gather(idxu_a, rows_a, gsem_a)
            pack(rows_a, sel_a)
            put(ua, sel_a, osem_a)
            wait_put(ua, sel_a, osem_a)

            @pl.when(2 * p + 2 < u_per_w)
            def _():
                stage(ua + 2, idxu_a)
                gather(idxu_a, rows_a, gsem_a)

            wait_gather(idxu_b, rows_b, gsem_b)
            pack(rows_b, sel_b)
            put(ub, sel_b, osem_b)
            wait_put(ub, sel_b, osem_b)

            @pl.when(2 * p + 3 < u_per_w)
            def _():
                stage(ub + 2, idxu_b)
                gather(idxu_b, rows_b, gsem_b)

            return 0

        lax.fori_loop(0, u_per_w // 2, body, 0)

    return k


@jax.jit
def kernel(input_sequences, weight):
    batch, hist = input_sequences.shape
    vocab, dim = weight.shape
    idx_t = input_sequences.T.astype(jnp.int32)
    table_p = jnp.pad(weight, ((0, 0), (0, dim)))
    fn = _build_gather(batch, hist, vocab, dim, weight.dtype.name)
    out5 = fn(idx_t, table_p)
    return jnp.transpose(out5, (2, 4, 0, 1, 3)).reshape(batch, hist, dim)
